# baseline (speedup 1.0000x reference)
import jax
import jax.numpy as jnp
from jax.experimental import pallas as pl
from jax.experimental.pallas import tpu as pltpu


def _gcn_kernel(flow_ref, edge_ref, w0_ref, b0_ref, w1_ref, b1_ref,
                w2_ref, b2_ref, out_ref):
    f = flow_ref[0]
    nrm = jnp.sqrt(jnp.sum(f * f, axis=1, keepdims=True))
    nx = f / jnp.maximum(nrm, 1e-12)
    r = jnp.sum(nx, axis=1, keepdims=True)
    deg = jax.lax.dot_general(nx, r, (((0,), (0,)), ((), ())),
                              preferred_element_type=jnp.float32) + 1.0
    dinv = jax.lax.rsqrt(deg)

    x = edge_ref[0]
    for w_ref, b_ref in ((w0_ref, b0_ref), (w1_ref, b1_ref), (w2_ref, b2_ref)):
        xw = jnp.dot(x, w_ref[...], preferred_element_type=jnp.float32)
        v = xw * dinv
        u = jnp.dot(nx, v, preferred_element_type=jnp.float32)
        y = jax.lax.dot_general(nx, u, (((0,), (0,)), ((), ())),
                                preferred_element_type=jnp.float32)
        x = jnp.maximum((y + v) * dinv + b_ref[...], 0.0)
    out_ref[0] = x


def kernel(Flow, Edge, W0, b0, W1, b1, W2, b2):
    batch, city, _, emb = Edge.shape
    T = Flow.shape[1]
    N = city * city
    flow2 = Flow.reshape(batch, T, N)
    edge2 = Edge.reshape(batch, N, emb)
    out = pl.pallas_call(
        _gcn_kernel,
        grid=(batch,),
        in_specs=[
            pl.BlockSpec((1, T, N), lambda b: (b, 0, 0)),
            pl.BlockSpec((1, N, emb), lambda b: (b, 0, 0)),
            pl.BlockSpec((emb, emb), lambda b: (0, 0)),
            pl.BlockSpec((1, emb), lambda b: (0, 0)),
            pl.BlockSpec((emb, emb), lambda b: (0, 0)),
            pl.BlockSpec((1, emb), lambda b: (0, 0)),
            pl.BlockSpec((emb, emb), lambda b: (0, 0)),
            pl.BlockSpec((1, emb), lambda b: (0, 0)),
        ],
        out_specs=pl.BlockSpec((1, N, emb), lambda b: (b, 0, 0)),
        out_shape=jax.ShapeDtypeStruct((batch, N, emb), jnp.float32),
        compiler_params=pltpu.CompilerParams(
            dimension_semantics=("parallel",)),
    )(flow2, edge2, W0, b0.reshape(1, emb), W1, b1.reshape(1, emb),
      W2, b2.reshape(1, emb))
    return out.reshape(batch, city, city, emb)


# speedup vs baseline: 1.6157x; 1.6157x over previous
import jax
import jax.numpy as jnp
from jax.experimental import pallas as pl
from jax.experimental.pallas import tpu as pltpu


def _gcn_kernel(flow_ref, edge_ref, w0_ref, b0_ref, w1_ref, b1_ref,
                w2_ref, b2_ref, out_ref):
    batch = flow_ref.shape[0]
    for bi in range(batch):
        f = flow_ref[bi]
        nrm = jnp.sqrt(jnp.sum(f * f, axis=1, keepdims=True))
        nx = f / jnp.maximum(nrm, 1e-12)
        r = jnp.sum(nx, axis=1, keepdims=True)
        deg = jax.lax.dot_general(nx, r, (((0,), (0,)), ((), ())),
                                  preferred_element_type=jnp.float32) + 1.0
        dinv = jax.lax.rsqrt(deg)

        x = edge_ref[bi]
        for w_ref, b_ref in ((w0_ref, b0_ref), (w1_ref, b1_ref),
                             (w2_ref, b2_ref)):
            xw = jnp.dot(x, w_ref[...], preferred_element_type=jnp.float32)
            v = xw * dinv
            u = jnp.dot(nx, v, preferred_element_type=jnp.float32)
            y = jax.lax.dot_general(nx, u, (((0,), (0,)), ((), ())),
                                    preferred_element_type=jnp.float32)
            x = jnp.maximum((y + v) * dinv + b_ref[...], 0.0)
        out_ref[bi] = x


def kernel(Flow, Edge, W0, b0, W1, b1, W2, b2):
    batch, city, _, emb = Edge.shape
    T = Flow.shape[1]
    N = city * city
    flow2 = Flow.reshape(batch, T, N)
    edge2 = Edge.reshape(batch, N, emb)
    out = pl.pallas_call(
        _gcn_kernel,
        out_shape=jax.ShapeDtypeStruct((batch, N, emb), jnp.float32),
        compiler_params=pltpu.CompilerParams(allow_input_fusion=[True] * 8),
    )(flow2, edge2, W0, b0.reshape(1, emb), W1, b1.reshape(1, emb),
      W2, b2.reshape(1, emb))
    return out.reshape(batch, city, city, emb)
